# baseline (speedup 1.0000x reference)
import functools

import jax
import jax.numpy as jnp
from jax import lax
from jax.experimental import pallas as pl
from jax.experimental.pallas import tpu as pltpu, tpu_sc as plsc

D = 1024
DW = D // 2

_info = plsc.get_sparse_core_info()
NC, NS, L = _info.num_cores, _info.num_subcores, _info.num_lanes
NW = NC * NS

B = 4 * 2048
B_PER_W = B // NW
CHUNK = 32
N_CHUNKS = B_PER_W // CHUNK

_mesh = plsc.VectorSubcoreMesh(core_axis_name="c", subcore_axis_name="s")


@functools.partial(
    pl.kernel,
    mesh=_mesh,
    out_type=jax.ShapeDtypeStruct((B * DW // 1024, 8, 128), jnp.int32),
    scratch_types=[
        pltpu.VMEM((2, 128), jnp.int32),
        pltpu.VMEM((2 * N_CHUNKS, 128), jnp.int32),
        pltpu.VMEM((2 * CHUNK * 4, 128), jnp.float32),
        pltpu.VMEM((2 * CHUNK * 4, 128), jnp.float32),
        pltpu.VMEM((16, 8, 128), jnp.int32),
        pltpu.VMEM((16, 8, 128), jnp.int32),
        pltpu.SemaphoreType.DMA,
        pltpu.SemaphoreType.DMA,
    ],
    compiler_params=pltpu.CompilerParams(
        use_tc_tiling_on_sc=False, needs_layout_passes=False),
)
def _embed_sc(idx_hbm, table_hbm, out_hbm, idx_v, item_v, rows0, rows1,
              outb0, outb1, gsem, osem):
    wid = lax.axis_index("s") * NC + lax.axis_index("c")
    pltpu.sync_copy(idx_hbm.at[pl.ds(2 * wid, 2)], idx_v)

    iota = lax.iota(jnp.int32, L)
    ev_lane = iota * 2

    for g in range(N_CHUNKS):
        for h in range(2):
            v = idx_v[g // 4, pl.ds(32 * (g % 4) + 16 * h, 16)]
            base = ((v >> 3) << 6) + (v & 7)
            for c in range(8):
                item_v[2 * g + h, pl.ds(16 * c, 16)] = base + 8 * c

    rows_bufs = (rows0, rows1)
    out_bufs = (outb0, outb1)

    def start_gather(g):
        buf = rows_bufs[g % 2]
        return (
            pltpu.async_copy(table_hbm.at[item_v.at[2 * g]],
                             buf.at[pl.ds(0, 128)], gsem),
            pltpu.async_copy(table_hbm.at[item_v.at[2 * g + 1]],
                             buf.at[pl.ds(128, 128)], gsem),
        )

    def convert_chunk(rows_ref, out_ref):
        def row_body(r, _):
            d0_base = ((r >> 4) << 7) + (r & 15)
            a_out = r >> 1
            b_par = (r & 1) * 4
            for j in range(32):
                d0 = lax.broadcast(d0_base + (j // 4) * 16, (L,))
                col = 32 * (j % 4)
                a = plsc.load_gather(rows_ref, [d0, col + ev_lane])
                b = plsc.load_gather(rows_ref, [d0, col + ev_lane + 1])
                ua = plsc.bitcast(a, jnp.int32)
                ub = plsc.bitcast(b, jnp.int32)
                ta = ua + 0x7FFF + ((ua >> 16) & 1)
                tb = ub + 0x7FFF + ((ub >> 16) & 1)
                word = (lax.shift_right_logical(ta, 16)
                        | (tb & jnp.int32(-0x10000)))
                out_ref[a_out, b_par + j // 8, pl.ds(16 * (j % 8), L)] = word
            return 0

        lax.fori_loop(0, CHUNK, row_body, 0)

    gh = [None] * N_CHUNKS
    oh = [None] * N_CHUNKS
    gh[0] = start_gather(0)
    for g in range(N_CHUNKS):
        gh[g][0].wait()
        gh[g][1].wait()
        if g + 1 < N_CHUNKS:
            gh[g + 1] = start_gather(g + 1)
        if g >= 2:
            oh[g - 2].wait()
        convert_chunk(rows_bufs[g % 2], out_bufs[g % 2])
        oh[g] = pltpu.async_copy(
            out_bufs[g % 2],
            out_hbm.at[pl.ds((wid * N_CHUNKS + g) * 16, 16)], osem)
    oh[N_CHUNKS - 2].wait()
    oh[N_CHUNKS - 1].wait()


def kernel(xBT, embedding):
    idx = xBT.reshape(64, 128)
    table = (embedding.reshape(12500, 8, 8, 128)
             .transpose(0, 2, 1, 3)
             .reshape(800000, 128))
    packed = _embed_sc(idx, table)
    out = lax.bitcast_convert_type(packed, jnp.bfloat16)
    return out.reshape(4, 2048, D)


# speedup vs baseline: 9.6654x; 9.6654x over previous
import functools

import jax
import jax.numpy as jnp
from jax.experimental import pallas as pl
from jax.experimental.pallas import tpu as pltpu

D = 1024
B = 4 * 2048
K = 16


def _gather_cast(idx_ref, *refs):
    in_refs = refs[:K]
    out_ref = refs[K]
    i = pl.program_id(0)
    for j in range(K):
        s = idx_ref[i * K + j] % 8
        out_ref[pl.ds(j, 1), :] = in_refs[j][pl.ds(s, 1), :].astype(
            jnp.bfloat16)


def _row_map(j, i, idx_ref):
    return (idx_ref[i * K + j] // 8, 0)


_grid_spec = pltpu.PrefetchScalarGridSpec(
    num_scalar_prefetch=1,
    grid=(B // K,),
    in_specs=[
        pl.BlockSpec((8, D), functools.partial(_row_map, j)) for j in range(K)
    ],
    out_specs=pl.BlockSpec((K, D), lambda i, idx_ref: (i, 0)),
)

_lookup = pl.pallas_call(
    _gather_cast,
    grid_spec=_grid_spec,
    out_shape=jax.ShapeDtypeStruct((B, D), jnp.bfloat16),
    compiler_params=pltpu.CompilerParams(
        dimension_semantics=("arbitrary",)),
)


def kernel(xBT, embedding):
    flat = xBT.reshape(B)
    out = _lookup(flat, *([embedding] * K))
    return out.reshape(4, 2048, D)


# speedup vs baseline: 11.9091x; 1.2321x over previous
import functools

import jax
import jax.numpy as jnp
from jax.experimental import pallas as pl
from jax.experimental.pallas import tpu as pltpu

D = 1024
B = 4 * 2048
K = 32


def _gather_cast(idx_ref, *refs):
    in_refs = refs[:K]
    out_ref = refs[K]
    i = pl.program_id(0)
    for j in range(K):
        s = idx_ref[i * K + j] % 8
        out_ref[pl.ds(j, 1), :] = in_refs[j][pl.ds(s, 1), :].astype(
            jnp.bfloat16)


def _row_map(j, i, idx_ref):
    return (idx_ref[i * K + j] // 8, 0)


_grid_spec = pltpu.PrefetchScalarGridSpec(
    num_scalar_prefetch=1,
    grid=(B // K,),
    in_specs=[
        pl.BlockSpec((8, D), functools.partial(_row_map, j)) for j in range(K)
    ],
    out_specs=pl.BlockSpec((K, D), lambda i, idx_ref: (i, 0)),
)

_lookup = pl.pallas_call(
    _gather_cast,
    grid_spec=_grid_spec,
    out_shape=jax.ShapeDtypeStruct((B, D), jnp.bfloat16),
    compiler_params=pltpu.CompilerParams(
        dimension_semantics=("arbitrary",)),
)


def kernel(xBT, embedding):
    flat = xBT.reshape(B)
    out = _lookup(flat, *([embedding] * K))
    return out.reshape(4, 2048, D)


# speedup vs baseline: 46.3089x; 3.8885x over previous
import jax
import jax.numpy as jnp
from jax import lax
from jax.experimental import pallas as pl
from jax.experimental.pallas import tpu as pltpu

D = 1024
B = 4 * 2048
G = 64
NG = B // G


def _gather_cast(idx_ref, table_ref, out_ref, inbuf, outstage,
                 gsem0, gsem1, osem0, osem1):
    gsems = (gsem0, gsem1)
    osems = (osem0, osem1)

    def issue_group(g, p):
        for j in range(G):
            r = idx_ref[g * G + j]
            pltpu.make_async_copy(
                table_ref.at[pl.ds(r, 1), :],
                inbuf.at[pl.ds(p * G + j, 1), :],
                gsems[p]).start()

    def wait_group(p):
        pltpu.make_async_copy(
            table_ref.at[pl.ds(0, G), :],
            inbuf.at[pl.ds(p * G, G), :],
            gsems[p]).wait()

    def out_dma(g, p):
        return pltpu.make_async_copy(
            outstage.at[pl.ds(p * G, G), :],
            out_ref.at[pl.ds(g * G, G), :],
            osems[p])

    def process_group(g, p):
        wait_group(p)
        for t in range(G // 8):
            outstage[pl.ds(p * G + 8 * t, 8), :] = (
                inbuf[pl.ds(p * G + 8 * t, 8), :].astype(jnp.bfloat16))
        out_dma(g, p).start()

    issue_group(0, 0)
    issue_group(1, 1)

    def body(i, _):
        g0 = 2 * i
        g1 = 2 * i + 1

        @pl.when(i >= 1)
        def _():
            out_dma(g0 - 2, 0).wait()

        process_group(g0, 0)

        @pl.when(i < NG // 2 - 1)
        def _():
            issue_group(g0 + 2, 0)

        @pl.when(i >= 1)
        def _():
            out_dma(g1 - 2, 1).wait()

        process_group(g1, 1)

        @pl.when(i < NG // 2 - 1)
        def _():
            issue_group(g1 + 2, 1)

        return 0

    lax.fori_loop(0, NG // 2, body, 0)
    out_dma(NG - 2, 0).wait()
    out_dma(NG - 1, 1).wait()


_grid_spec = pltpu.PrefetchScalarGridSpec(
    num_scalar_prefetch=1,
    grid=(1,),
    in_specs=[pl.BlockSpec(memory_space=pl.ANY)],
    out_specs=pl.BlockSpec(memory_space=pl.ANY),
    scratch_shapes=[
        pltpu.VMEM((2 * G, D), jnp.float32),
        pltpu.VMEM((2 * G, D), jnp.bfloat16),
        pltpu.SemaphoreType.DMA,
        pltpu.SemaphoreType.DMA,
        pltpu.SemaphoreType.DMA,
        pltpu.SemaphoreType.DMA,
    ],
)

_lookup = pl.pallas_call(
    _gather_cast,
    grid_spec=_grid_spec,
    out_shape=jax.ShapeDtypeStruct((B, D), jnp.bfloat16),
    compiler_params=pltpu.CompilerParams(
        dimension_semantics=("arbitrary",)),
)


def kernel(xBT, embedding):
    flat = xBT.reshape(B)
    out = _lookup(flat, embedding)
    return out.reshape(4, 2048, D)


# speedup vs baseline: 70.3042x; 1.5182x over previous
import jax
import jax.numpy as jnp
from jax import lax
from jax.experimental import pallas as pl
from jax.experimental.pallas import tpu as pltpu

D = 1024
B = 4 * 2048
G = 128
NG = B // G


def _gather_cast(idx_ref, table_ref, out_ref, inbuf, outstage,
                 gsem0, gsem1, osem0, osem1):
    gsems = (gsem0, gsem1)
    osems = (osem0, osem1)

    def issue_group(g, p):
        for j in range(G):
            r = idx_ref[g * G + j]
            pltpu.make_async_copy(
                table_ref.at[pl.ds(r, 1), :],
                inbuf.at[pl.ds(p * G + j, 1), :],
                gsems[p]).start()

    def wait_group(p):
        pltpu.make_async_copy(
            table_ref.at[pl.ds(0, G), :],
            inbuf.at[pl.ds(p * G, G), :],
            gsems[p]).wait()

    def out_dma(g, p):
        return pltpu.make_async_copy(
            outstage.at[pl.ds(p * G, G), :],
            out_ref.at[pl.ds(g * G, G), :],
            osems[p])

    def process_group(g, p):
        wait_group(p)
        for t in range(G // 8):
            outstage[pl.ds(p * G + 8 * t, 8), :] = (
                inbuf[pl.ds(p * G + 8 * t, 8), :].astype(jnp.bfloat16))
        out_dma(g, p).start()

    issue_group(0, 0)
    issue_group(1, 1)

    def body(i, _):
        g0 = 2 * i
        g1 = 2 * i + 1

        @pl.when(i >= 1)
        def _():
            out_dma(g0 - 2, 0).wait()

        process_group(g0, 0)

        @pl.when(i < NG // 2 - 1)
        def _():
            issue_group(g0 + 2, 0)

        @pl.when(i >= 1)
        def _():
            out_dma(g1 - 2, 1).wait()

        process_group(g1, 1)

        @pl.when(i < NG // 2 - 1)
        def _():
            issue_group(g1 + 2, 1)

        return 0


    lax.fori_loop(0, NG // 2, body, 0)
    out_dma(NG - 2, 0).wait()
    out_dma(NG - 1, 1).wait()


_grid_spec = pltpu.PrefetchScalarGridSpec(
    num_scalar_prefetch=1,
    grid=(1,),
    in_specs=[pl.BlockSpec(memory_space=pl.ANY)],
    out_specs=pl.BlockSpec(memory_space=pl.ANY),
    scratch_shapes=[
        pltpu.VMEM((2 * G, D), jnp.float32),
        pltpu.VMEM((2 * G, D), jnp.bfloat16),
        pltpu.SemaphoreType.DMA,
        pltpu.SemaphoreType.DMA,
        pltpu.SemaphoreType.DMA,
        pltpu.SemaphoreType.DMA,
    ],
)

_lookup = pl.pallas_call(
    _gather_cast,
    grid_spec=_grid_spec,
    out_shape=jax.ShapeDtypeStruct((B, D), jnp.bfloat16),
    compiler_params=pltpu.CompilerParams(
        dimension_semantics=("arbitrary",)),
)


def kernel(xBT, embedding):
    flat = xBT.reshape(B)
    out = _lookup(flat, embedding)
    return out.reshape(4, 2048, D)


# speedup vs baseline: 98.9138x; 1.4069x over previous
import jax
import jax.numpy as jnp
from jax import lax
from jax.experimental import pallas as pl
from jax.experimental.pallas import tpu as pltpu

D = 1024
B = 4 * 2048
G = 256
NG = B // G


def _gather_cast(idx_ref, table_ref, out_ref, inbuf, outstage,
                 gsem0, gsem1, osem0, osem1):
    gsems = (gsem0, gsem1)
    osems = (osem0, osem1)

    def issue_group(g, p):
        for j in range(G):
            r = idx_ref[g * G + j]
            pltpu.make_async_copy(
                table_ref.at[pl.ds(r, 1), :],
                inbuf.at[pl.ds(p * G + j, 1), :],
                gsems[p]).start()

    def wait_group(p):
        pltpu.make_async_copy(
            table_ref.at[pl.ds(0, G), :],
            inbuf.at[pl.ds(p * G, G), :],
            gsems[p]).wait()

    def out_dma(g, p):
        return pltpu.make_async_copy(
            outstage.at[pl.ds(p * G, G), :],
            out_ref.at[pl.ds(g * G, G), :],
            osems[p])

    def process_group(g, p):
        wait_group(p)
        for t in range(G // 8):
            outstage[pl.ds(p * G + 8 * t, 8), :] = (
                inbuf[pl.ds(p * G + 8 * t, 8), :].astype(jnp.bfloat16))
        out_dma(g, p).start()

    issue_group(0, 0)
    issue_group(1, 1)

    def body(i, _):
        g0 = 2 * i
        g1 = 2 * i + 1

        @pl.when(i >= 1)
        def _():
            out_dma(g0 - 2, 0).wait()

        process_group(g0, 0)

        @pl.when(i < NG // 2 - 1)
        def _():
            issue_group(g0 + 2, 0)

        @pl.when(i >= 1)
        def _():
            out_dma(g1 - 2, 1).wait()

        process_group(g1, 1)

        @pl.when(i < NG // 2 - 1)
        def _():
            issue_group(g1 + 2, 1)

        return 0


    lax.fori_loop(0, NG // 2, body, 0)
    out_dma(NG - 2, 0).wait()
    out_dma(NG - 1, 1).wait()


_grid_spec = pltpu.PrefetchScalarGridSpec(
    num_scalar_prefetch=1,
    grid=(1,),
    in_specs=[pl.BlockSpec(memory_space=pl.ANY)],
    out_specs=pl.BlockSpec(memory_space=pl.ANY),
    scratch_shapes=[
        pltpu.VMEM((2 * G, D), jnp.float32),
        pltpu.VMEM((2 * G, D), jnp.bfloat16),
        pltpu.SemaphoreType.DMA,
        pltpu.SemaphoreType.DMA,
        pltpu.SemaphoreType.DMA,
        pltpu.SemaphoreType.DMA,
    ],
)

_lookup = pl.pallas_call(
    _gather_cast,
    grid_spec=_grid_spec,
    out_shape=jax.ShapeDtypeStruct((B, D), jnp.bfloat16),
    compiler_params=pltpu.CompilerParams(
        dimension_semantics=("arbitrary",)),
)


def kernel(xBT, embedding):
    flat = xBT.reshape(B)
    out = _lookup(flat, embedding)
    return out.reshape(4, 2048, D)
